# speedup vs baseline: 2.3537x; 2.3537x over previous
import jax
import jax.numpy as jnp
from jax.experimental import pallas as pl


def _add_block(x_ref, pe_ref, o_ref):
    o_ref[...] = x_ref[...] + pe_ref[...]


@jax.jit
def _pe_add(x, position_embeddings):
    seq_len, d_model = x.shape
    block_rows = 512
    grid = (seq_len // block_rows,)
    return pl.pallas_call(
        _add_block,
        grid=grid,
        in_specs=[
            pl.BlockSpec((block_rows, d_model), lambda i: (i, 0)),
            pl.BlockSpec((block_rows, d_model), lambda i: (i, 0)),
        ],
        out_specs=pl.BlockSpec((block_rows, d_model), lambda i: (i, 0)),
        out_shape=jax.ShapeDtypeStruct((seq_len, d_model), x.dtype),
    )(x, position_embeddings)


def kernel(x, position_embeddings):
    seq_len = x.shape[0]
    return _pe_add(x, position_embeddings[:seq_len])


# speedup vs baseline: 2.4185x; 1.0275x over previous
import jax
import jax.numpy as jnp
from jax.experimental import pallas as pl


def _add_block(x_ref, pe_ref, o_ref):
    o_ref[...] = x_ref[...] + pe_ref[...]


@jax.jit
def _pe_add(x, position_embeddings):
    seq_len, d_model = x.shape
    block_rows = 1024
    grid = (seq_len // block_rows,)
    return pl.pallas_call(
        _add_block,
        grid=grid,
        in_specs=[
            pl.BlockSpec((block_rows, d_model), lambda i: (i, 0)),
            pl.BlockSpec((block_rows, d_model), lambda i: (i, 0)),
        ],
        out_specs=pl.BlockSpec((block_rows, d_model), lambda i: (i, 0)),
        out_shape=jax.ShapeDtypeStruct((seq_len, d_model), x.dtype),
    )(x, position_embeddings)


def kernel(x, position_embeddings):
    seq_len = x.shape[0]
    return _pe_add(x, position_embeddings[:seq_len])
